# baseline (speedup 1.0000x reference)
import functools

import jax
import jax.numpy as jnp
from jax import lax
from jax.experimental import pallas as pl
from jax.experimental.pallas import tpu as pltpu
from jax.experimental.pallas import tpu_sc as plsc

N_IN = 100000
C = 128
K = 16
H = 32
N_OUT = 50000

NC = 2
NS = 16
NW = NC * NS

N_PER_W = 1600
N_PAD = NW * N_PER_W
CHUNK_OUT = 64
ROWS_PER_CHUNK = CHUNK_OUT * K
N_CHUNKS = N_PER_W // CHUNK_OUT
GATHER_SPLIT = 128


def _xw_body(x_ref, w_ref, o_ref):
    acc = jnp.dot(x_ref[0].astype(jnp.bfloat16), w_ref[...],
                  preferred_element_type=jnp.float32)
    half = acc.shape[1] // 2
    lo = jax.lax.bitcast_convert_type(
        acc[:, :half].astype(jnp.bfloat16).astype(jnp.float32), jnp.int32)
    hi = jax.lax.bitcast_convert_type(
        acc[:, half:].astype(jnp.bfloat16).astype(jnp.float32), jnp.int32)
    packed = jax.lax.shift_right_logical(lo, 16) | (hi & jnp.int32(-65536))
    o_ref[...] = packed.reshape(o_ref.shape)


def _gelu_exact(v):
    return 0.5 * v * (1.0 + lax.erf(v * (2.0 ** -0.5)))


def _mlp_body(s_ref, b0_ref, w1_ref, b1_ref, w2_ref, b2_ref, o_ref):
    h0 = _gelu_exact(s_ref[...] + b0_ref[...])
    h1 = jnp.dot(h0, w1_ref[...], preferred_element_type=jnp.float32) + b1_ref[...]
    h1 = _gelu_exact(h1)
    o_ref[...] = jnp.dot(h1, w2_ref[...],
                         preferred_element_type=jnp.float32) + b2_ref[...]


N_STREAMS = ROWS_PER_CHUNK // GATHER_SPLIT
N_PAIRS = (N_CHUNKS - 1) // 2


def _gather_sum_body(xw_hbm, conn_hbm, out_hbm, conn_v, idx_v, rows_v, acc_v,
                     sem_c0, sem_c1, sem_g0, sem_g1, sem_o0, sem_o1):
    wid = lax.axis_index("s") * NC + lax.axis_index("c")
    base_out = wid * N_PER_W
    kvec = lax.iota(jnp.int32, 16)
    sem_c, sem_g, sem_o = (sem_c0, sem_c1), (sem_g0, sem_g1), (sem_o0, sem_o1)

    def out0_of(c):
        return jnp.minimum(base_out + c * CHUNK_OUT, N_OUT - CHUNK_OUT)

    def fire_conn(c, b):
        pltpu.async_copy(conn_hbm.at[pl.ds(out0_of(c) * K, ROWS_PER_CHUNK)],
                         conn_v.at[b], sem_c[b])

    def wait_conn(b):
        pltpu.make_async_copy(conn_hbm.at[pl.ds(0, ROWS_PER_CHUNK)],
                              conn_v.at[b], sem_c[b]).wait()

    def idx_and_gather(b):
        def idx_body(j, carry):
            idx_v[b, pl.ds(j * K, K)] = conn_v[b, pl.ds(j * K, K)] * K + kvec
            return carry

        lax.fori_loop(0, CHUNK_OUT, idx_body, 0, unroll=4)
        for g in range(N_STREAMS):
            pltpu.async_copy(
                xw_hbm.at[idx_v.at[b, pl.ds(g * GATHER_SPLIT, GATHER_SPLIT)]],
                rows_v.at[b, pl.ds(g * GATHER_SPLIT, GATHER_SPLIT)],
                sem_g[b])

    def wait_gather(b):
        pltpu.make_async_copy(xw_hbm.at[pl.ds(0, ROWS_PER_CHUNK), :],
                              rows_v.at[b], sem_g[b]).wait()

    def wait_out(b):
        pltpu.make_async_copy(acc_v.at[b],
                              out_hbm.at[pl.ds(0, CHUNK_OUT // 4), :],
                              sem_o[b]).wait()

    def reduce_and_store(c, b):
        def out_body(j, carry):
            r0 = j * K
            a0 = None
            for r in range(K):
                words = plsc.bitcast(rows_v[b, r0 + r, :], jnp.bfloat16)
                a, bb = plsc.unpack(words, format=plsc.PackFormat.INTERLEAVED,
                                    preferred_element_type=jnp.float32)
                a0 = a if a0 is None else a0 + a
                b0_ = bb if r == 0 else b0_ + bb
            lane0 = (j % 4) * H
            acc_v[b, j // 4, pl.ds(lane0, 16)] = a0
            acc_v[b, j // 4, pl.ds(lane0 + 16, 16)] = b0_
            return carry

        lax.fori_loop(0, CHUNK_OUT, out_body, 0, unroll=2)
        pltpu.async_copy(
            acc_v.at[b],
            out_hbm.at[pl.ds(out0_of(c) // 4, CHUNK_OUT // 4), :],
            sem_o[b])

    fire_conn(0, 0)
    wait_conn(0)
    idx_and_gather(0)
    fire_conn(1, 1)

    def pair_body(t, carry):
        c0 = 2 * t
        wait_conn(1)
        idx_and_gather(1)
        fire_conn(c0 + 2, 0)
        wait_gather(0)

        @pl.when(t > 0)
        def _():
            wait_out(0)

        reduce_and_store(c0, 0)
        wait_conn(0)
        idx_and_gather(0)

        @pl.when(t < N_PAIRS - 1)
        def _():
            fire_conn(c0 + 3, 1)

        wait_gather(1)

        @pl.when(t > 0)
        def _():
            wait_out(1)

        reduce_and_store(c0 + 1, 1)
        return carry

    lax.fori_loop(0, N_PAIRS, pair_body, 0)

    wait_gather(0)
    wait_out(0)
    reduce_and_store(N_CHUNKS - 1, 0)
    wait_out(0)
    wait_out(1)


@functools.cache
def _gather_sum():
    return functools.partial(
        pl.kernel,
        out_type=jax.ShapeDtypeStruct((N_OUT * H // 128, 128), jnp.float32),
        mesh=plsc.VectorSubcoreMesh(core_axis_name="c", subcore_axis_name="s",
                                    num_cores=NC, num_subcores=NS),
        scratch_types=[
            pltpu.VMEM((2, ROWS_PER_CHUNK), jnp.int32),
            pltpu.VMEM((2, ROWS_PER_CHUNK), jnp.int32),
            pltpu.VMEM((2, ROWS_PER_CHUNK, H // 2), jnp.int32),
            pltpu.VMEM((2, CHUNK_OUT // 4, 128), jnp.float32),
            pltpu.SemaphoreType.DMA,
            pltpu.SemaphoreType.DMA,
            pltpu.SemaphoreType.DMA,
            pltpu.SemaphoreType.DMA,
            pltpu.SemaphoreType.DMA,
            pltpu.SemaphoreType.DMA,
        ],
        compiler_params=pltpu.CompilerParams(use_tc_tiling_on_sc=False,
                                             needs_layout_passes=False),
    )(_gather_sum_body)


def kernel(x, connection_indices, W0, b0, W1, b1, W2, b2):
    B = x.shape[0]

    w0r = W0.reshape(K, C, H).transpose(1, 0, 2)
    w0m = jnp.concatenate([w0r[:, :, :16], w0r[:, :, 16:]], axis=1)
    w0m = w0m.reshape(C, K * H).astype(jnp.bfloat16)

    blk = 2000
    xw = pl.pallas_call(
        _xw_body,
        grid=(N_IN // blk,),
        in_specs=[
            pl.BlockSpec((1, blk, C), lambda i: (0, i, 0)),
            pl.BlockSpec((C, K * H), lambda i: (0, 0)),
        ],
        out_specs=pl.BlockSpec((blk * K * H // 2,), lambda i: (i,)),
        out_shape=jax.ShapeDtypeStruct((N_IN * K * H // 2,), jnp.int32),
    )(x, w0m)
    xw_flat = xw.reshape(N_IN * K, H // 2)

    s = _gather_sum()(xw_flat, connection_indices.reshape(N_OUT * K))

    blk2 = 512
    eye4 = jnp.eye(4, dtype=jnp.float32)
    w1bd = jnp.kron(eye4, W1)
    w2bd = jnp.kron(eye4, W2)
    b0r = jnp.tile(b0, 4).reshape(1, 128)
    b1r = jnp.tile(b1, 4).reshape(1, 128)
    b2r = jnp.tile(b2, 4).reshape(1, 128)
    rows = N_OUT * H // 128
    out = pl.pallas_call(
        _mlp_body,
        grid=((rows + blk2 - 1) // blk2,),
        in_specs=[
            pl.BlockSpec((blk2, 128), lambda i: (i, 0)),
            pl.BlockSpec((1, 128), lambda i: (0, 0)),
            pl.BlockSpec((128, 128), lambda i: (0, 0)),
            pl.BlockSpec((1, 128), lambda i: (0, 0)),
            pl.BlockSpec((128, 128), lambda i: (0, 0)),
            pl.BlockSpec((1, 128), lambda i: (0, 0)),
        ],
        out_specs=pl.BlockSpec((blk2, 128), lambda i: (i, 0)),
        out_shape=jax.ShapeDtypeStruct((rows, 128), jnp.float32),
    )(s, b0r, w1bd, b1r, w2bd, b2r)

    return out.reshape(B, N_OUT, H)


# speedup vs baseline: 1.0762x; 1.0762x over previous
import functools

import jax
import jax.numpy as jnp
from jax import lax
from jax.experimental import pallas as pl
from jax.experimental.pallas import tpu as pltpu
from jax.experimental.pallas import tpu_sc as plsc

N_IN = 100000
C = 128
K = 16
H = 32
N_OUT = 50000

NC = 2
NS = 16
NW = NC * NS

N_PER_W = 1600
N_PAD = NW * N_PER_W
CHUNK_OUT = 64
ROWS_PER_CHUNK = CHUNK_OUT * K
N_CHUNKS = N_PER_W // CHUNK_OUT
GATHER_SPLIT = 128


def _xw_body(x_ref, w_ref, o_ref):
    acc = jnp.dot(x_ref[0].astype(jnp.bfloat16), w_ref[...],
                  preferred_element_type=jnp.float32)
    half = acc.shape[1] // 2
    lo = jax.lax.bitcast_convert_type(
        acc[:, :half].astype(jnp.bfloat16).astype(jnp.float32), jnp.int32)
    hi = jax.lax.bitcast_convert_type(
        acc[:, half:].astype(jnp.bfloat16).astype(jnp.float32), jnp.int32)
    packed = jax.lax.shift_right_logical(lo, 16) | (hi & jnp.int32(-65536))
    o_ref[...] = packed.reshape(o_ref.shape)


def _gelu_exact(v):
    return 0.5 * v * (1.0 + lax.erf(v * (2.0 ** -0.5)))


def _mlp_body(s_ref, b0_ref, w1_ref, b1_ref, w2_ref, b2_ref, o_ref):
    h0 = _gelu_exact(s_ref[...] + b0_ref[...])
    h1 = jnp.dot(h0, w1_ref[...], preferred_element_type=jnp.float32) + b1_ref[...]
    h1 = _gelu_exact(h1)
    o_ref[...] = jnp.dot(h1, w2_ref[...],
                         preferred_element_type=jnp.float32) + b2_ref[...]


N_STREAMS = ROWS_PER_CHUNK // GATHER_SPLIT
N_PAIRS = (N_CHUNKS - 1) // 2


def _gather_sum_body(xw_hbm, conn_hbm, out_hbm, conn_v, idx_v, rows_v, acc_v,
                     sem_c0, sem_c1, sem_g0, sem_g1, sem_o0, sem_o1):
    wid = lax.axis_index("s") * NC + lax.axis_index("c")
    base_out = wid * N_PER_W
    kvec = lax.iota(jnp.int32, 16)
    sem_c, sem_g, sem_o = (sem_c0, sem_c1), (sem_g0, sem_g1), (sem_o0, sem_o1)

    def out0_of(c):
        return jnp.minimum(base_out + c * CHUNK_OUT, N_OUT - CHUNK_OUT)

    def fire_conn(c, b):
        pltpu.async_copy(conn_hbm.at[pl.ds(out0_of(c) * K, ROWS_PER_CHUNK)],
                         conn_v.at[b], sem_c[b])

    def wait_conn(b):
        pltpu.make_async_copy(conn_hbm.at[pl.ds(0, ROWS_PER_CHUNK)],
                              conn_v.at[b], sem_c[b]).wait()

    def idx_and_gather(b):
        def idx_body(j, carry):
            idx_v[b, pl.ds(j * K, K)] = conn_v[b, pl.ds(j * K, K)] * K + kvec
            return carry

        lax.fori_loop(0, CHUNK_OUT, idx_body, 0, unroll=4)
        for g in range(N_STREAMS):
            pltpu.async_copy(
                xw_hbm.at[idx_v.at[b, pl.ds(g * GATHER_SPLIT, GATHER_SPLIT)]],
                rows_v.at[b, pl.ds(g * GATHER_SPLIT, GATHER_SPLIT)],
                sem_g[b])

    def wait_gather(b):
        pltpu.make_async_copy(xw_hbm.at[pl.ds(0, ROWS_PER_CHUNK), :],
                              rows_v.at[b], sem_g[b]).wait()

    def wait_out(b):
        pltpu.make_async_copy(acc_v.at[b],
                              out_hbm.at[pl.ds(0, CHUNK_OUT // 4), :],
                              sem_o[b]).wait()

    def reduce_and_store(c, b):
        def out_body(j, carry):
            r0 = j * K
            a0 = None
            for r in range(K):
                words = plsc.bitcast(rows_v[b, r0 + r, :], jnp.bfloat16)
                a, bb = plsc.unpack(words, format=plsc.PackFormat.INTERLEAVED,
                                    preferred_element_type=jnp.float32)
                a0 = a if a0 is None else a0 + a
                b0_ = bb if r == 0 else b0_ + bb
            lane0 = (j % 4) * H
            acc_v[b, j // 4, pl.ds(lane0, 16)] = a0
            acc_v[b, j // 4, pl.ds(lane0 + 16, 16)] = b0_
            return carry

        lax.fori_loop(0, CHUNK_OUT, out_body, 0, unroll=2)
        pltpu.async_copy(
            acc_v.at[b],
            out_hbm.at[pl.ds(out0_of(c) // 4, CHUNK_OUT // 4), :],
            sem_o[b])

    fire_conn(0, 0)
    wait_conn(0)
    idx_and_gather(0)
    fire_conn(1, 1)

    def pair_body(t, carry):
        c0 = 2 * t
        wait_conn(1)
        idx_and_gather(1)
        fire_conn(c0 + 2, 0)
        wait_gather(0)

        @pl.when(t > 0)
        def _():
            wait_out(0)

        reduce_and_store(c0, 0)
        wait_conn(0)
        idx_and_gather(0)

        @pl.when(t < N_PAIRS - 1)
        def _():
            fire_conn(c0 + 3, 1)

        wait_gather(1)

        @pl.when(t > 0)
        def _():
            wait_out(1)

        reduce_and_store(c0 + 1, 1)
        return carry

    lax.fori_loop(0, N_PAIRS, pair_body, 0)

    wait_gather(0)
    wait_out(0)
    reduce_and_store(N_CHUNKS - 1, 0)
    wait_out(0)
    wait_out(1)


@functools.cache
def _gather_sum():
    return functools.partial(
        pl.kernel,
        out_type=jax.ShapeDtypeStruct((N_OUT * H // 128, 128), jnp.float32),
        mesh=plsc.VectorSubcoreMesh(core_axis_name="c", subcore_axis_name="s",
                                    num_cores=NC, num_subcores=NS),
        scratch_types=[
            pltpu.VMEM((2, ROWS_PER_CHUNK), jnp.int32),
            pltpu.VMEM((2, ROWS_PER_CHUNK), jnp.int32),
            pltpu.VMEM((2, ROWS_PER_CHUNK, H // 2), jnp.int32),
            pltpu.VMEM((2, CHUNK_OUT // 4, 128), jnp.float32),
            pltpu.SemaphoreType.DMA,
            pltpu.SemaphoreType.DMA,
            pltpu.SemaphoreType.DMA,
            pltpu.SemaphoreType.DMA,
            pltpu.SemaphoreType.DMA,
            pltpu.SemaphoreType.DMA,
        ],
        compiler_params=pltpu.CompilerParams(use_tc_tiling_on_sc=False,
                                             needs_layout_passes=False),
    )(_gather_sum_body)


def kernel(x, connection_indices, W0, b0, W1, b1, W2, b2):
    B = x.shape[0]

    w0r = W0.reshape(K, C, H).transpose(1, 0, 2)
    w0m = jnp.concatenate([w0r[:, :, :16], w0r[:, :, 16:]], axis=1)
    w0m = w0m.reshape(C, K * H).astype(jnp.bfloat16)

    blk = 4000
    xw = pl.pallas_call(
        _xw_body,
        grid=(N_IN // blk,),
        in_specs=[
            pl.BlockSpec((1, blk, C), lambda i: (0, i, 0)),
            pl.BlockSpec((C, K * H), lambda i: (0, 0)),
        ],
        out_specs=pl.BlockSpec((blk * K * H // 2,), lambda i: (i,)),
        out_shape=jax.ShapeDtypeStruct((N_IN * K * H // 2,), jnp.int32),
    )(x, w0m)
    xw_flat = xw.reshape(N_IN * K, H // 2)

    s = _gather_sum()(xw_flat, connection_indices.reshape(N_OUT * K))

    blk2 = 512
    eye4 = jnp.eye(4, dtype=jnp.float32)
    w1bd = jnp.kron(eye4, W1)
    w2bd = jnp.kron(eye4, W2)
    b0r = jnp.tile(b0, 4).reshape(1, 128)
    b1r = jnp.tile(b1, 4).reshape(1, 128)
    b2r = jnp.tile(b2, 4).reshape(1, 128)
    rows = N_OUT * H // 128
    out = pl.pallas_call(
        _mlp_body,
        grid=((rows + blk2 - 1) // blk2,),
        in_specs=[
            pl.BlockSpec((blk2, 128), lambda i: (i, 0)),
            pl.BlockSpec((1, 128), lambda i: (0, 0)),
            pl.BlockSpec((128, 128), lambda i: (0, 0)),
            pl.BlockSpec((1, 128), lambda i: (0, 0)),
            pl.BlockSpec((128, 128), lambda i: (0, 0)),
            pl.BlockSpec((1, 128), lambda i: (0, 0)),
        ],
        out_specs=pl.BlockSpec((blk2, 128), lambda i: (i, 0)),
        out_shape=jax.ShapeDtypeStruct((rows, 128), jnp.float32),
    )(s, b0r, w1bd, b1r, w2bd, b2r)

    return out.reshape(B, N_OUT, H)


# speedup vs baseline: 1.1096x; 1.0310x over previous
import functools

import jax
import jax.numpy as jnp
from jax import lax
from jax.experimental import pallas as pl
from jax.experimental.pallas import tpu as pltpu
from jax.experimental.pallas import tpu_sc as plsc

N_IN = 100000
C = 128
K = 16
H = 32
N_OUT = 50000

NC = 2
NS = 16
NW = NC * NS

N_PER_W = 1600
N_PAD = NW * N_PER_W
CHUNK_OUT = 64
ROWS_PER_CHUNK = CHUNK_OUT * K
N_CHUNKS = N_PER_W // CHUNK_OUT
GATHER_SPLIT = 128


def _xw_body(x_ref, w_ref, o_ref):
    acc = jnp.dot(x_ref[0].astype(jnp.bfloat16), w_ref[...],
                  preferred_element_type=jnp.float32)
    half = acc.shape[1] // 2
    lo = jax.lax.bitcast_convert_type(
        acc[:, :half].astype(jnp.bfloat16).astype(jnp.float32), jnp.int32)
    hi = jax.lax.bitcast_convert_type(
        acc[:, half:].astype(jnp.bfloat16).astype(jnp.float32), jnp.int32)
    packed = jax.lax.shift_right_logical(lo, 16) | (hi & jnp.int32(-65536))
    o_ref[...] = packed.reshape(o_ref.shape)


def _gelu_exact(v):
    return 0.5 * v * (1.0 + lax.erf(v * (2.0 ** -0.5)))


def _mlp_body(s_ref, b0_ref, w1_ref, b1_ref, w2_ref, b2_ref, o_ref):
    h0 = _gelu_exact(s_ref[...] + b0_ref[...])
    h1 = jnp.dot(h0, w1_ref[...], preferred_element_type=jnp.float32) + b1_ref[...]
    h1 = _gelu_exact(h1)
    o_ref[...] = jnp.dot(h1, w2_ref[...],
                         preferred_element_type=jnp.float32) + b2_ref[...]


N_STREAMS = ROWS_PER_CHUNK // GATHER_SPLIT
N_PAIRS = (N_CHUNKS - 1) // 2


def _gather_sum_body(xw_hbm, conn_hbm, out_hbm, conn_v, idx_v, rows_v, acc_v,
                     sem_c0, sem_c1, sem_g0, sem_g1, sem_o0, sem_o1):
    wid = lax.axis_index("s") * NC + lax.axis_index("c")
    base_out = wid * N_PER_W
    kvec = lax.iota(jnp.int32, 16)
    sem_c, sem_g, sem_o = (sem_c0, sem_c1), (sem_g0, sem_g1), (sem_o0, sem_o1)

    def out0_of(c):
        return jnp.minimum(base_out + c * CHUNK_OUT, N_OUT - CHUNK_OUT)

    def fire_conn(c, b):
        pltpu.async_copy(conn_hbm.at[pl.ds(out0_of(c) * K, ROWS_PER_CHUNK)],
                         conn_v.at[b], sem_c[b])

    def wait_conn(b):
        pltpu.make_async_copy(conn_hbm.at[pl.ds(0, ROWS_PER_CHUNK)],
                              conn_v.at[b], sem_c[b]).wait()

    def idx_and_gather(b):
        def idx_body(j, carry):
            idx_v[b, pl.ds(j * K, K)] = conn_v[b, pl.ds(j * K, K)] * K + kvec
            return carry

        lax.fori_loop(0, CHUNK_OUT, idx_body, 0, unroll=4)
        for g in range(N_STREAMS):
            pltpu.async_copy(
                xw_hbm.at[idx_v.at[b, pl.ds(g * GATHER_SPLIT, GATHER_SPLIT)]],
                rows_v.at[b, pl.ds(g * GATHER_SPLIT, GATHER_SPLIT)],
                sem_g[b])

    def wait_gather(b):
        pltpu.make_async_copy(xw_hbm.at[pl.ds(0, ROWS_PER_CHUNK), :],
                              rows_v.at[b], sem_g[b]).wait()

    def wait_out(b):
        pltpu.make_async_copy(acc_v.at[b],
                              out_hbm.at[pl.ds(0, CHUNK_OUT // 4), :],
                              sem_o[b]).wait()

    def reduce_and_store(c, b):
        def out_body(j, carry):
            r0 = j * K
            a0 = None
            for r in range(K):
                words = plsc.bitcast(rows_v[b, r0 + r, :], jnp.bfloat16)
                a, bb = plsc.unpack(words, format=plsc.PackFormat.INTERLEAVED,
                                    preferred_element_type=jnp.float32)
                a0 = a if a0 is None else a0 + a
                b0_ = bb if r == 0 else b0_ + bb
            lane0 = (j % 4) * H
            acc_v[b, j // 4, pl.ds(lane0, 16)] = a0
            acc_v[b, j // 4, pl.ds(lane0 + 16, 16)] = b0_
            return carry

        lax.fori_loop(0, CHUNK_OUT, out_body, 0, unroll=2)
        pltpu.async_copy(
            acc_v.at[b],
            out_hbm.at[pl.ds(out0_of(c) // 4, CHUNK_OUT // 4), :],
            sem_o[b])

    fire_conn(0, 0)
    wait_conn(0)
    idx_and_gather(0)
    fire_conn(1, 1)

    def pair_body(t, carry):
        c0 = 2 * t
        wait_conn(1)
        idx_and_gather(1)
        fire_conn(c0 + 2, 0)
        wait_gather(0)

        @pl.when(t > 0)
        def _():
            wait_out(0)

        reduce_and_store(c0, 0)
        wait_conn(0)
        idx_and_gather(0)

        @pl.when(t < N_PAIRS - 1)
        def _():
            fire_conn(c0 + 3, 1)

        wait_gather(1)

        @pl.when(t > 0)
        def _():
            wait_out(1)

        reduce_and_store(c0 + 1, 1)
        return carry

    lax.fori_loop(0, N_PAIRS, pair_body, 0)

    wait_gather(0)
    wait_out(0)
    reduce_and_store(N_CHUNKS - 1, 0)
    wait_out(0)
    wait_out(1)


@functools.cache
def _gather_sum():
    return functools.partial(
        pl.kernel,
        out_type=jax.ShapeDtypeStruct((N_OUT * H // 128, 128), jnp.float32),
        mesh=plsc.VectorSubcoreMesh(core_axis_name="c", subcore_axis_name="s",
                                    num_cores=NC, num_subcores=NS),
        scratch_types=[
            pltpu.VMEM((2, ROWS_PER_CHUNK), jnp.int32),
            pltpu.VMEM((2, ROWS_PER_CHUNK), jnp.int32),
            pltpu.VMEM((2, ROWS_PER_CHUNK, H // 2), jnp.int32),
            pltpu.VMEM((2, CHUNK_OUT // 4, 128), jnp.float32),
            pltpu.SemaphoreType.DMA,
            pltpu.SemaphoreType.DMA,
            pltpu.SemaphoreType.DMA,
            pltpu.SemaphoreType.DMA,
            pltpu.SemaphoreType.DMA,
            pltpu.SemaphoreType.DMA,
        ],
        compiler_params=pltpu.CompilerParams(use_tc_tiling_on_sc=False,
                                             needs_layout_passes=False),
    )(_gather_sum_body)


def kernel(x, connection_indices, W0, b0, W1, b1, W2, b2):
    B = x.shape[0]

    w0r = W0.reshape(K, C, H).transpose(1, 0, 2)
    w0m = jnp.concatenate([w0r[:, :, :16], w0r[:, :, 16:]], axis=1)
    w0m = w0m.reshape(C, K * H).astype(jnp.bfloat16)

    blk = 10000
    xw = pl.pallas_call(
        _xw_body,
        grid=(N_IN // blk,),
        in_specs=[
            pl.BlockSpec((1, blk, C), lambda i: (0, i, 0)),
            pl.BlockSpec((C, K * H), lambda i: (0, 0)),
        ],
        out_specs=pl.BlockSpec((blk * K * H // 2,), lambda i: (i,)),
        out_shape=jax.ShapeDtypeStruct((N_IN * K * H // 2,), jnp.int32),
    )(x, w0m)
    xw_flat = xw.reshape(N_IN * K, H // 2)

    s = _gather_sum()(xw_flat, connection_indices.reshape(N_OUT * K))

    blk2 = 512
    eye4 = jnp.eye(4, dtype=jnp.float32)
    w1bd = jnp.kron(eye4, W1)
    w2bd = jnp.kron(eye4, W2)
    b0r = jnp.tile(b0, 4).reshape(1, 128)
    b1r = jnp.tile(b1, 4).reshape(1, 128)
    b2r = jnp.tile(b2, 4).reshape(1, 128)
    rows = N_OUT * H // 128
    out = pl.pallas_call(
        _mlp_body,
        grid=((rows + blk2 - 1) // blk2,),
        in_specs=[
            pl.BlockSpec((blk2, 128), lambda i: (i, 0)),
            pl.BlockSpec((1, 128), lambda i: (0, 0)),
            pl.BlockSpec((128, 128), lambda i: (0, 0)),
            pl.BlockSpec((1, 128), lambda i: (0, 0)),
            pl.BlockSpec((128, 128), lambda i: (0, 0)),
            pl.BlockSpec((1, 128), lambda i: (0, 0)),
        ],
        out_specs=pl.BlockSpec((blk2, 128), lambda i: (i, 0)),
        out_shape=jax.ShapeDtypeStruct((rows, 128), jnp.float32),
    )(s, b0r, w1bd, b1r, w2bd, b2r)

    return out.reshape(B, N_OUT, H)
